# baseline (speedup 1.0000x reference)
import functools

import jax
import jax.numpy as jnp
import numpy as np
from jax import lax
from jax.experimental import pallas as pl
from jax.experimental.pallas import tpu as pltpu
from jax.experimental.pallas import tpu_sc as plsc

B = 2
C = 256
NH = 8
DH = 32
NL = 4
NP = 4
D_FFN = 1024
SHAPES = np.array([[128, 128], [64, 64], [32, 32], [16, 16]], dtype=np.int64)
AREAS = SHAPES[:, 0] * SHAPES[:, 1]
NQ = int(AREAS.sum())
LSI = np.concatenate([[0], np.cumsum(AREAS)[:-1]]).astype(np.int64)
BQ = B * NQ
NROWS = BQ * NH

QB1 = 256
GRID1 = BQ // QB1

_K = np.arange(128)
_H_OF_K = _K // 16
_L_OF_K = (_K % 16) // 4
_WV = SHAPES[_L_OF_K, 1].astype(np.float32)
_HV = SHAPES[_L_OF_K, 0].astype(np.float32)
_W8V = (SHAPES[_L_OF_K, 1] * NH).astype(np.int32)
_BASE8 = (LSI[_L_OF_K] * NH + _H_OF_K).astype(np.int32)
_G = (_K[:, None] // 16 == _K[None, :] // 16).astype(np.float32)


def _k1_body(src_ref, pos_ref, rpb_ref, wval_ref, bval_ref, woff_ref, boff_ref,
             wattn_ref, battn_ref, g_ref, cwf_ref, chf_ref, ciw8_ref, cbase_ref,
             val_ref, idx_ref, w_ref):
    i = pl.program_id(0)
    b = i // (GRID1 // B)
    src = src_ref[...]
    q = src + pos_ref[...]
    val_ref[...] = (
        jnp.dot(src, wval_ref[...], preferred_element_type=jnp.float32)
        + bval_ref[...])
    off = (jnp.dot(q, woff_ref[...], preferred_element_type=jnp.float32)
           + boff_ref[...])
    logit = (jnp.dot(q, wattn_ref[...], preferred_element_type=jnp.float32)
             + battn_ref[...])
    e = jnp.exp(logit)
    aw = e / jnp.dot(e, g_ref[...], preferred_element_type=jnp.float32)

    rpb = rpb_ref[...]
    cw = cwf_ref[...]
    ch = chf_ref[...]
    xs = rpb[:, :128] * cw + off[:, :128] - 0.5
    ys = rpb[:, 128:] * ch + off[:, 128:] - 0.5
    x0f = jnp.floor(xs)
    y0f = jnp.floor(ys)
    fx = xs - x0f
    fy = ys - y0f
    wmax = cw - 1.0
    hmax = ch - 1.0
    x1f = x0f + 1.0
    y1f = y0f + 1.0
    vx0 = ((x0f >= 0.0) & (x0f <= wmax)).astype(jnp.float32)
    vx1 = ((x1f >= 0.0) & (x1f <= wmax)).astype(jnp.float32)
    vy0 = ((y0f >= 0.0) & (y0f <= hmax)).astype(jnp.float32)
    vy1 = ((y1f >= 0.0) & (y1f <= hmax)).astype(jnp.float32)
    x0c = jnp.clip(x0f, 0.0, wmax).astype(jnp.int32)
    x1c = jnp.clip(x1f, 0.0, wmax).astype(jnp.int32)
    y0c = jnp.clip(y0f, 0.0, hmax).astype(jnp.int32)
    y1c = jnp.clip(y1f, 0.0, hmax).astype(jnp.int32)

    w8 = ciw8_ref[...]
    cb = cbase_ref[...]
    bofs = b * (NQ * NH)
    r00 = bofs + cb + y0c * w8 + x0c * NH
    r10 = bofs + cb + y0c * w8 + x1c * NH
    r01 = bofs + cb + y1c * w8 + x0c * NH
    r11 = bofs + cb + y1c * w8 + x1c * NH
    idx_ref[...] = jnp.concatenate([r00, r10, r01, r11], axis=1)

    gx0 = (1.0 - fx) * vx0
    gx1 = fx * vx1
    gy0 = (1.0 - fy) * vy0
    gy1 = fy * vy1
    w_ref[...] = jnp.concatenate(
        [aw * gx0 * gy0, aw * gx1 * gy0, aw * gx0 * gy1, aw * gx1 * gy1],
        axis=1)


def _ln(x, g, b):
    m = jnp.mean(x, axis=-1, keepdims=True)
    d = x - m
    v = jnp.mean(d * d, axis=-1, keepdims=True)
    return d * lax.rsqrt(v + 1e-5) * g + b


def _k2_body(smp_ref, src_ref, wout_ref, bout_ref, w1_ref, b1_ref, w2_ref,
             b2_ref, g1_ref, be1_ref, g2_ref, be2_ref, out_ref):
    s0 = (jnp.dot(smp_ref[...], wout_ref[...], preferred_element_type=jnp.float32)
          + bout_ref[...] + src_ref[...])
    x1 = _ln(s0, g1_ref[...], be1_ref[...])
    h = jnp.maximum(
        jnp.dot(x1, w1_ref[...], preferred_element_type=jnp.float32) + b1_ref[...],
        0.0)
    ff = jnp.dot(h, w2_ref[...], preferred_element_type=jnp.float32) + b2_ref[...]
    out_ref[...] = _ln(x1 + ff, g2_ref[...], be2_ref[...])


NW = 32
QPW = BQ // NW


QC = 2
NCH = QPW // QC


def _sc_body(table_hbm, idx_hbm, w_hbm, out_hbm,
             idxA, idxB, wA, wB, rowsA, rowsB, outv,
             semGA, semGB, semIA, semIB, semWA, semWB):
    cid = lax.axis_index("c")
    sid = lax.axis_index("s")
    wid = sid * 2 + cid
    cbase = wid * NCH

    def fire(idxX, rowsX, semX):
        for qs in range(QC):
            for cc in range(4):
                pltpu.async_copy(
                    table_hbm.at[idxX.at[qs, cc]],
                    rowsX.at[pl.ds(qs * 512 + cc * 128, 128)], semX)

    def drain_rows(rowsX, semX):
        pltpu.make_async_copy(
            table_hbm.at[pl.ds(0, QC * 512)], rowsX, semX).wait()

    def drain_idx(idxX, semX):
        pltpu.make_async_copy(idx_hbm.at[pl.ds(0, QC)], idxX, semX).wait()

    def drain_w(wX, semX):
        pltpu.make_async_copy(w_hbm.at[pl.ds(0, QC)], wX, semX).wait()

    def compute(g, wX, rowsX):
        for qs in range(QC):
            for h in range(NH):
                zer = jnp.zeros((16,), jnp.float32)
                a0e, a0o, a1e, a1o = zer, zer, zer, zer
                for cc in range(4):
                    off = cc * 128 + h * 16
                    wvec = wX[qs, pl.ds(off, 16)]
                    rbase = qs * 512 + off
                    for t in range(16):
                        ws = lax.gather(
                            wvec, jnp.full((16, 1), t, jnp.int32),
                            lax.GatherDimensionNumbers(
                                offset_dims=(), collapsed_slice_dims=(0,),
                                start_index_map=(0,)),
                            (1,), mode=lax.GatherScatterMode.PROMISE_IN_BOUNDS)
                        lo = ws * rowsX[rbase + t, 0:16]
                        hi = ws * rowsX[rbase + t, 16:32]
                        if t % 2 == 0:
                            a0e, a1e = a0e + lo, a1e + hi
                        else:
                            a0o, a1o = a0o + lo, a1o + hi
                outv[qs * NH + h, 0:16] = a0e + a0o
                outv[qs * NH + h, 16:32] = a1e + a1o
        pltpu.sync_copy(
            outv, out_hbm.at[pl.ds((cbase + g) * (QC * NH), QC * NH)])

    q0 = cbase * QC
    pltpu.sync_copy(idx_hbm.at[pl.ds(q0, QC)], idxA)
    fire(idxA, rowsA, semGA)
    pltpu.async_copy(idx_hbm.at[pl.ds(q0 + QC, QC)], idxB, semIB)
    pltpu.async_copy(w_hbm.at[pl.ds(q0, QC)], wA, semWA)
    pltpu.async_copy(w_hbm.at[pl.ds(q0 + QC, QC)], wB, semWB)

    bufsA = (idxA, wA, rowsA, semGA, semIA, semWA)
    bufsB = (idxB, wB, rowsB, semGB, semIB, semWB)

    def pair(i, carry):
        for par, (iX, wX, rX, sGX, sIX, sWX), (iY, wY, rY, sGY, sIY, sWY) in (
                (0, bufsA, bufsB), (1, bufsB, bufsA)):
            g = 2 * i + par
            drain_idx(iY, sIY)
            fire(iY, rY, sGY)
            qpre = (cbase + jnp.minimum(g + 2, NCH - 1)) * QC
            pltpu.async_copy(idx_hbm.at[pl.ds(qpre, QC)], iX, sIX)
            drain_rows(rX, sGX)
            drain_w(wX, sWX)
            compute(g, wX, rX)
            pltpu.async_copy(w_hbm.at[pl.ds(qpre, QC)], wX, sWX)
        return carry

    lax.fori_loop(0, NCH // 2, pair, 0)

    drain_rows(rowsA, semGA)
    drain_idx(idxB, semIB)
    drain_w(wA, semWA)
    drain_w(wB, semWB)


def _sample_sc(table, idx3, wts):
    return pl.kernel(
        _sc_body,
        out_type=jax.ShapeDtypeStruct((NROWS, DH), jnp.float32),
        mesh=plsc.VectorSubcoreMesh(core_axis_name="c", subcore_axis_name="s"),
        scratch_types=[
            pltpu.VMEM((QC, 4, 128), jnp.int32),
            pltpu.VMEM((QC, 4, 128), jnp.int32),
            pltpu.VMEM((QC, 512), jnp.float32),
            pltpu.VMEM((QC, 512), jnp.float32),
            pltpu.VMEM((QC * 512, DH), jnp.float32),
            pltpu.VMEM((QC * 512, DH), jnp.float32),
            pltpu.VMEM((QC * NH, DH), jnp.float32),
            pltpu.SemaphoreType.DMA,
            pltpu.SemaphoreType.DMA,
            pltpu.SemaphoreType.DMA,
            pltpu.SemaphoreType.DMA,
            pltpu.SemaphoreType.DMA,
            pltpu.SemaphoreType.DMA,
        ],
        compiler_params=pltpu.CompilerParams(use_tc_tiling_on_sc=False),
    )(table, idx3, wts)


def _stage1_call(src2, pos2, rpb, wval, bval, woffp, boffp, wattn, battn):
    consts_f = [jnp.asarray(_G), jnp.asarray(_WV[None, :]),
                jnp.asarray(_HV[None, :])]
    consts_i = [jnp.asarray(_W8V[None, :]), jnp.asarray(_BASE8[None, :])]
    row_spec = pl.BlockSpec((QB1, 256), lambda i: (i, 0))
    full = lambda shape: pl.BlockSpec(shape, lambda i: (0,) * len(shape))
    return pl.pallas_call(
        _k1_body,
        grid=(GRID1,),
        in_specs=[
            row_spec, row_spec, row_spec,
            full((C, C)), full((1, C)),
            full((C, C)), full((1, C)),
            full((C, 128)), full((1, 128)),
            full((128, 128)), full((1, 128)), full((1, 128)),
            full((1, 128)), full((1, 128)),
        ],
        out_specs=[
            pl.BlockSpec((QB1, C), lambda i: (i, 0)),
            pl.BlockSpec((QB1, 512), lambda i: (i, 0)),
            pl.BlockSpec((QB1, 512), lambda i: (i, 0)),
        ],
        out_shape=[
            jax.ShapeDtypeStruct((BQ, C), jnp.float32),
            jax.ShapeDtypeStruct((BQ, 512), jnp.int32),
            jax.ShapeDtypeStruct((BQ, 512), jnp.float32),
        ],
    )(src2, pos2, rpb, wval, bval, woffp, boffp, wattn, battn, *consts_f,
      *consts_i)


def _stage2_call(smp2, src2, wout, bout, w1, b1, w2, b2, g1, be1, g2, be2):
    row_spec = pl.BlockSpec((QB1, 256), lambda i: (i, 0))
    full = lambda shape: pl.BlockSpec(shape, lambda i: (0,) * len(shape))
    return pl.pallas_call(
        _k2_body,
        grid=(GRID1,),
        in_specs=[
            row_spec, row_spec,
            full((C, C)), full((1, C)),
            full((C, D_FFN)), full((1, D_FFN)),
            full((D_FFN, C)), full((1, C)),
            full((1, C)), full((1, C)), full((1, C)), full((1, C)),
        ],
        out_specs=pl.BlockSpec((QB1, C), lambda i: (i, 0)),
        out_shape=jax.ShapeDtypeStruct((BQ, C), jnp.float32),
    )(smp2, src2, wout, bout, w1, b1, w2, b2, g1, be1, g2, be2)


def kernel(src, pos, reference_points, spatial_shapes, level_start_index,
           padding_mask, W_off, b_off, W_attn, b_attn, W_val, b_val, W_out,
           b_out, W1, b1, W2, b2, g1, be1, g2, be2):
    del spatial_shapes, level_start_index, padding_mask

    src2 = src.reshape(BQ, C)
    pos2 = pos.reshape(BQ, C)

    rp = reference_points
    rpx = jnp.broadcast_to(rp[:, :, None, :, None, 0], (B, NQ, NH, NL, NP))
    rpy = jnp.broadcast_to(rp[:, :, None, :, None, 1], (B, NQ, NH, NL, NP))
    rpb = jnp.concatenate(
        [rpx.reshape(BQ, 128), rpy.reshape(BQ, 128)], axis=-1)

    woffp = W_off.reshape(C, NH, NL, NP, 2).transpose(0, 4, 1, 2, 3).reshape(C, 256)
    boffp = b_off.reshape(NH, NL, NP, 2).transpose(3, 0, 1, 2).reshape(1, 256)

    val, idx, wts = _stage1_call(
        src2, pos2, rpb, W_val, b_val.reshape(1, C), woffp, boffp, W_attn,
        b_attn.reshape(1, 128))

    table = val.reshape(NROWS, DH)
    idx3 = idx.reshape(BQ, 4, 128)
    sampled = _sample_sc(table, idx3, wts).reshape(BQ, C)

    out = _stage2_call(
        sampled, src2, W_out, b_out.reshape(1, C), W1, b1.reshape(1, D_FFN),
        W2, b2.reshape(1, C), g1.reshape(1, C), be1.reshape(1, C),
        g2.reshape(1, C), be2.reshape(1, C))
    return out.reshape(B, NQ, C)


# speedup vs baseline: 1.0624x; 1.0624x over previous
import functools

import jax
import jax.numpy as jnp
import numpy as np
from jax import lax
from jax.experimental import pallas as pl
from jax.experimental.pallas import tpu as pltpu
from jax.experimental.pallas import tpu_sc as plsc

B = 2
C = 256
NH = 8
DH = 32
NL = 4
NP = 4
D_FFN = 1024
SHAPES = np.array([[128, 128], [64, 64], [32, 32], [16, 16]], dtype=np.int64)
AREAS = SHAPES[:, 0] * SHAPES[:, 1]
NQ = int(AREAS.sum())
LSI = np.concatenate([[0], np.cumsum(AREAS)[:-1]]).astype(np.int64)
BQ = B * NQ
NROWS = BQ * NH

QB1 = 256
GRID1 = BQ // QB1

_K = np.arange(128)
_H_OF_K = _K // 16
_L_OF_K = (_K % 16) // 4
_WV = SHAPES[_L_OF_K, 1].astype(np.float32)
_HV = SHAPES[_L_OF_K, 0].astype(np.float32)
_W8V = (SHAPES[_L_OF_K, 1] * NH).astype(np.int32)
_BASE8 = (LSI[_L_OF_K] * NH + _H_OF_K).astype(np.int32)
_G = (_K[:, None] // 16 == _K[None, :] // 16).astype(np.float32)


def _k1_body(src_ref, pos_ref, rpb_ref, wval_ref, bval_ref, woff_ref, boff_ref,
             wattn_ref, battn_ref, g_ref, cwf_ref, chf_ref, ciw8_ref, cbase_ref,
             val_ref, idx_ref, w_ref):
    i = pl.program_id(0)
    b = i // (GRID1 // B)
    src = src_ref[...]
    q = src + pos_ref[...]
    val_ref[...] = (
        jnp.dot(src, wval_ref[...], preferred_element_type=jnp.float32)
        + bval_ref[...])
    off = (jnp.dot(q, woff_ref[...], preferred_element_type=jnp.float32)
           + boff_ref[...])
    logit = (jnp.dot(q, wattn_ref[...], preferred_element_type=jnp.float32)
             + battn_ref[...])
    e = jnp.exp(logit)
    aw = e / jnp.dot(e, g_ref[...], preferred_element_type=jnp.float32)

    rpb = rpb_ref[...]
    cw = cwf_ref[...]
    ch = chf_ref[...]
    xs = rpb[:, :128] * cw + off[:, :128] - 0.5
    ys = rpb[:, 128:] * ch + off[:, 128:] - 0.5
    x0f = jnp.floor(xs)
    y0f = jnp.floor(ys)
    fx = xs - x0f
    fy = ys - y0f
    wmax = cw - 1.0
    hmax = ch - 1.0
    x1f = x0f + 1.0
    y1f = y0f + 1.0
    vx0 = ((x0f >= 0.0) & (x0f <= wmax)).astype(jnp.float32)
    vx1 = ((x1f >= 0.0) & (x1f <= wmax)).astype(jnp.float32)
    vy0 = ((y0f >= 0.0) & (y0f <= hmax)).astype(jnp.float32)
    vy1 = ((y1f >= 0.0) & (y1f <= hmax)).astype(jnp.float32)
    x0c = jnp.clip(x0f, 0.0, wmax).astype(jnp.int32)
    x1c = jnp.clip(x1f, 0.0, wmax).astype(jnp.int32)
    y0c = jnp.clip(y0f, 0.0, hmax).astype(jnp.int32)
    y1c = jnp.clip(y1f, 0.0, hmax).astype(jnp.int32)

    w8 = ciw8_ref[...]
    cb = cbase_ref[...]
    bofs = b * (NQ * NH)
    r00 = bofs + cb + y0c * w8 + x0c * NH
    r10 = bofs + cb + y0c * w8 + x1c * NH
    r01 = bofs + cb + y1c * w8 + x0c * NH
    r11 = bofs + cb + y1c * w8 + x1c * NH
    idx_ref[...] = jnp.concatenate([r00, r10, r01, r11], axis=1)

    gx0 = (1.0 - fx) * vx0
    gx1 = fx * vx1
    gy0 = (1.0 - fy) * vy0
    gy1 = fy * vy1
    w_ref[...] = jnp.concatenate(
        [aw * gx0 * gy0, aw * gx1 * gy0, aw * gx0 * gy1, aw * gx1 * gy1],
        axis=1)


def _ln(x, g, b):
    m = jnp.mean(x, axis=-1, keepdims=True)
    d = x - m
    v = jnp.mean(d * d, axis=-1, keepdims=True)
    return d * lax.rsqrt(v + 1e-5) * g + b


def _k2_body(smp_ref, src_ref, wout_ref, bout_ref, w1_ref, b1_ref, w2_ref,
             b2_ref, g1_ref, be1_ref, g2_ref, be2_ref, out_ref):
    s0 = (jnp.dot(smp_ref[...], wout_ref[...], preferred_element_type=jnp.float32)
          + bout_ref[...] + src_ref[...])
    x1 = _ln(s0, g1_ref[...], be1_ref[...])
    h = jnp.maximum(
        jnp.dot(x1, w1_ref[...], preferred_element_type=jnp.float32) + b1_ref[...],
        0.0)
    ff = jnp.dot(h, w2_ref[...], preferred_element_type=jnp.float32) + b2_ref[...]
    out_ref[...] = _ln(x1 + ff, g2_ref[...], be2_ref[...])


NW = 32
QPW = BQ // NW


QC = 2
NCH = QPW // QC


def _sc_body(table_hbm, idx_hbm, w_hbm, out_hbm,
             idxA, idxB, wA, wB, rowsA, rowsB, outv,
             semGA, semGB, semIA, semIB, semWA, semWB):
    cid = lax.axis_index("c")
    sid = lax.axis_index("s")
    wid = sid * 2 + cid
    cbase = wid * NCH

    def fire(idxX, rowsX, semX):
        for qs in range(QC):
            for cc in range(4):
                pltpu.async_copy(
                    table_hbm.at[idxX.at[qs, cc]],
                    rowsX.at[pl.ds(qs * 512 + cc * 128, 128)], semX)

    def drain_rows(rowsX, semX):
        pltpu.make_async_copy(
            table_hbm.at[pl.ds(0, QC * 512)], rowsX, semX).wait()

    def drain_idx(idxX, semX):
        pltpu.make_async_copy(idx_hbm.at[pl.ds(0, QC)], idxX, semX).wait()

    def drain_w(wX, semX):
        pltpu.make_async_copy(w_hbm.at[pl.ds(0, QC)], wX, semX).wait()

    def compute(g, wX, rowsX):
        for qs in range(QC):
            for h in range(NH):
                zer = jnp.zeros((16,), jnp.float32)
                a0e, a0o, a1e, a1o = zer, zer, zer, zer
                for cc in range(4):
                    off = cc * 128 + h * 16
                    wvec = wX[qs, pl.ds(off, 16)]
                    rbase = qs * 512 + off
                    wss = [wvec[t] for t in range(16)]
                    for t in range(16):
                        lo = wss[t] * rowsX[rbase + t, 0:16]
                        hi = wss[t] * rowsX[rbase + t, 16:32]
                        if t % 2 == 0:
                            a0e, a1e = a0e + lo, a1e + hi
                        else:
                            a0o, a1o = a0o + lo, a1o + hi
                outv[qs * NH + h, 0:16] = a0e + a0o
                outv[qs * NH + h, 16:32] = a1e + a1o
        pltpu.sync_copy(
            outv, out_hbm.at[pl.ds((cbase + g) * (QC * NH), QC * NH)])

    q0 = cbase * QC
    pltpu.sync_copy(idx_hbm.at[pl.ds(q0, QC)], idxA)
    fire(idxA, rowsA, semGA)
    pltpu.async_copy(idx_hbm.at[pl.ds(q0 + QC, QC)], idxB, semIB)
    pltpu.async_copy(w_hbm.at[pl.ds(q0, QC)], wA, semWA)
    pltpu.async_copy(w_hbm.at[pl.ds(q0 + QC, QC)], wB, semWB)

    bufsA = (idxA, wA, rowsA, semGA, semIA, semWA)
    bufsB = (idxB, wB, rowsB, semGB, semIB, semWB)

    def pair(i, carry):
        for par, (iX, wX, rX, sGX, sIX, sWX), (iY, wY, rY, sGY, sIY, sWY) in (
                (0, bufsA, bufsB), (1, bufsB, bufsA)):
            g = 2 * i + par
            drain_idx(iY, sIY)
            fire(iY, rY, sGY)
            qpre = (cbase + jnp.minimum(g + 2, NCH - 1)) * QC
            pltpu.async_copy(idx_hbm.at[pl.ds(qpre, QC)], iX, sIX)
            drain_rows(rX, sGX)
            drain_w(wX, sWX)
            compute(g, wX, rX)
            pltpu.async_copy(w_hbm.at[pl.ds(qpre, QC)], wX, sWX)
        return carry

    lax.fori_loop(0, NCH // 2, pair, 0)

    drain_rows(rowsA, semGA)
    drain_idx(idxB, semIB)
    drain_w(wA, semWA)
    drain_w(wB, semWB)


def _sample_sc(table, idx3, wts):
    return pl.kernel(
        _sc_body,
        out_type=jax.ShapeDtypeStruct((NROWS, DH), jnp.float32),
        mesh=plsc.VectorSubcoreMesh(core_axis_name="c", subcore_axis_name="s"),
        scratch_types=[
            pltpu.VMEM((QC, 4, 128), jnp.int32),
            pltpu.VMEM((QC, 4, 128), jnp.int32),
            pltpu.VMEM((QC, 512), jnp.float32),
            pltpu.VMEM((QC, 512), jnp.float32),
            pltpu.VMEM((QC * 512, DH), jnp.float32),
            pltpu.VMEM((QC * 512, DH), jnp.float32),
            pltpu.VMEM((QC * NH, DH), jnp.float32),
            pltpu.SemaphoreType.DMA,
            pltpu.SemaphoreType.DMA,
            pltpu.SemaphoreType.DMA,
            pltpu.SemaphoreType.DMA,
            pltpu.SemaphoreType.DMA,
            pltpu.SemaphoreType.DMA,
        ],
        compiler_params=pltpu.CompilerParams(use_tc_tiling_on_sc=False),
    )(table, idx3, wts)


def _stage1_call(src2, pos2, rpb, wval, bval, woffp, boffp, wattn, battn):
    consts_f = [jnp.asarray(_G), jnp.asarray(_WV[None, :]),
                jnp.asarray(_HV[None, :])]
    consts_i = [jnp.asarray(_W8V[None, :]), jnp.asarray(_BASE8[None, :])]
    row_spec = pl.BlockSpec((QB1, 256), lambda i: (i, 0))
    full = lambda shape: pl.BlockSpec(shape, lambda i: (0,) * len(shape))
    return pl.pallas_call(
        _k1_body,
        grid=(GRID1,),
        in_specs=[
            row_spec, row_spec, row_spec,
            full((C, C)), full((1, C)),
            full((C, C)), full((1, C)),
            full((C, 128)), full((1, 128)),
            full((128, 128)), full((1, 128)), full((1, 128)),
            full((1, 128)), full((1, 128)),
        ],
        out_specs=[
            pl.BlockSpec((QB1, C), lambda i: (i, 0)),
            pl.BlockSpec((QB1, 512), lambda i: (i, 0)),
            pl.BlockSpec((QB1, 512), lambda i: (i, 0)),
        ],
        out_shape=[
            jax.ShapeDtypeStruct((BQ, C), jnp.float32),
            jax.ShapeDtypeStruct((BQ, 512), jnp.int32),
            jax.ShapeDtypeStruct((BQ, 512), jnp.float32),
        ],
    )(src2, pos2, rpb, wval, bval, woffp, boffp, wattn, battn, *consts_f,
      *consts_i)


def _stage2_call(smp2, src2, wout, bout, w1, b1, w2, b2, g1, be1, g2, be2):
    row_spec = pl.BlockSpec((QB1, 256), lambda i: (i, 0))
    full = lambda shape: pl.BlockSpec(shape, lambda i: (0,) * len(shape))
    return pl.pallas_call(
        _k2_body,
        grid=(GRID1,),
        in_specs=[
            row_spec, row_spec,
            full((C, C)), full((1, C)),
            full((C, D_FFN)), full((1, D_FFN)),
            full((D_FFN, C)), full((1, C)),
            full((1, C)), full((1, C)), full((1, C)), full((1, C)),
        ],
        out_specs=pl.BlockSpec((QB1, C), lambda i: (i, 0)),
        out_shape=jax.ShapeDtypeStruct((BQ, C), jnp.float32),
    )(smp2, src2, wout, bout, w1, b1, w2, b2, g1, be1, g2, be2)


def kernel(src, pos, reference_points, spatial_shapes, level_start_index,
           padding_mask, W_off, b_off, W_attn, b_attn, W_val, b_val, W_out,
           b_out, W1, b1, W2, b2, g1, be1, g2, be2):
    del spatial_shapes, level_start_index, padding_mask

    src2 = src.reshape(BQ, C)
    pos2 = pos.reshape(BQ, C)

    rp = reference_points
    rpx = jnp.broadcast_to(rp[:, :, None, :, None, 0], (B, NQ, NH, NL, NP))
    rpy = jnp.broadcast_to(rp[:, :, None, :, None, 1], (B, NQ, NH, NL, NP))
    rpb = jnp.concatenate(
        [rpx.reshape(BQ, 128), rpy.reshape(BQ, 128)], axis=-1)

    woffp = W_off.reshape(C, NH, NL, NP, 2).transpose(0, 4, 1, 2, 3).reshape(C, 256)
    boffp = b_off.reshape(NH, NL, NP, 2).transpose(3, 0, 1, 2).reshape(1, 256)

    val, idx, wts = _stage1_call(
        src2, pos2, rpb, W_val, b_val.reshape(1, C), woffp, boffp, W_attn,
        b_attn.reshape(1, 128))

    table = val.reshape(NROWS, DH)
    idx3 = idx.reshape(BQ, 4, 128)
    sampled = _sample_sc(table, idx3, wts).reshape(BQ, C)

    out = _stage2_call(
        sampled, src2, W_out, b_out.reshape(1, C), W1, b1.reshape(1, D_FFN),
        W2, b2.reshape(1, C), g1.reshape(1, C), be1.reshape(1, C),
        g2.reshape(1, C), be2.reshape(1, C))
    return out.reshape(B, NQ, C)


# speedup vs baseline: 1.0986x; 1.0341x over previous
import functools

import jax
import jax.numpy as jnp
import numpy as np
from jax import lax
from jax.experimental import pallas as pl
from jax.experimental.pallas import tpu as pltpu
from jax.experimental.pallas import tpu_sc as plsc

B = 2
C = 256
NH = 8
DH = 32
NL = 4
NP = 4
D_FFN = 1024
SHAPES = np.array([[128, 128], [64, 64], [32, 32], [16, 16]], dtype=np.int64)
AREAS = SHAPES[:, 0] * SHAPES[:, 1]
NQ = int(AREAS.sum())
LSI = np.concatenate([[0], np.cumsum(AREAS)[:-1]]).astype(np.int64)
BQ = B * NQ
NROWS = NQ * NH

QB1 = 256
GRID1 = NQ // QB1

_K = np.arange(128)
_H_OF_K = _K // 16
_L_OF_K = (_K % 16) // 4
_WV = SHAPES[_L_OF_K, 1].astype(np.float32)
_HV = SHAPES[_L_OF_K, 0].astype(np.float32)
_W8V = (SHAPES[_L_OF_K, 1] * NH).astype(np.int32)
_BASE8 = (LSI[_L_OF_K] * NH + _H_OF_K).astype(np.int32)
_G = (_K[:, None] // 16 == _K[None, :] // 16).astype(np.float32)


def _k1_body(src_ref, pos_ref, rpb_ref, wval_ref, bval_ref, woff_ref, boff_ref,
             wattn_ref, battn_ref, g_ref, cwf_ref, chf_ref, ciw8_ref, cbase_ref,
             val_ref, idx_ref, w_ref):
    src = src_ref[...]
    q = src + pos_ref[...]
    val_ref[...] = (
        jnp.dot(src, wval_ref[...], preferred_element_type=jnp.float32)
        + bval_ref[...])
    off = (jnp.dot(q, woff_ref[...], preferred_element_type=jnp.float32)
           + boff_ref[...])
    logit = (jnp.dot(q, wattn_ref[...], preferred_element_type=jnp.float32)
             + battn_ref[...])
    e = jnp.exp(logit)
    aw = e / jnp.dot(e, g_ref[...], preferred_element_type=jnp.float32)

    rpb = rpb_ref[...]
    cw = cwf_ref[...]
    ch = chf_ref[...]
    xs = rpb[:, :128] * cw + off[:, :128] - 0.5
    ys = rpb[:, 128:] * ch + off[:, 128:] - 0.5
    x0f = jnp.floor(xs)
    y0f = jnp.floor(ys)
    fx = xs - x0f
    fy = ys - y0f
    wmax = cw - 1.0
    hmax = ch - 1.0
    x1f = x0f + 1.0
    y1f = y0f + 1.0
    vx0 = ((x0f >= 0.0) & (x0f <= wmax)).astype(jnp.float32)
    vx1 = ((x1f >= 0.0) & (x1f <= wmax)).astype(jnp.float32)
    vy0 = ((y0f >= 0.0) & (y0f <= hmax)).astype(jnp.float32)
    vy1 = ((y1f >= 0.0) & (y1f <= hmax)).astype(jnp.float32)
    x0c = jnp.clip(x0f, 0.0, wmax).astype(jnp.int32)
    x1c = jnp.clip(x1f, 0.0, wmax).astype(jnp.int32)
    y0c = jnp.clip(y0f, 0.0, hmax).astype(jnp.int32)
    y1c = jnp.clip(y1f, 0.0, hmax).astype(jnp.int32)

    w8 = ciw8_ref[...]
    cb = cbase_ref[...]
    r00 = cb + y0c * w8 + x0c * NH
    r10 = cb + y0c * w8 + x1c * NH
    r01 = cb + y1c * w8 + x0c * NH
    r11 = cb + y1c * w8 + x1c * NH
    idx_ref[...] = jnp.concatenate([r00, r10, r01, r11], axis=1)

    gx0 = (1.0 - fx) * vx0
    gx1 = fx * vx1
    gy0 = (1.0 - fy) * vy0
    gy1 = fy * vy1
    w_ref[...] = jnp.concatenate(
        [aw * gx0 * gy0, aw * gx1 * gy0, aw * gx0 * gy1, aw * gx1 * gy1],
        axis=1)


def _ln(x, g, b):
    m = jnp.mean(x, axis=-1, keepdims=True)
    d = x - m
    v = jnp.mean(d * d, axis=-1, keepdims=True)
    return d * lax.rsqrt(v + 1e-5) * g + b


def _k2_body(smp_ref, src_ref, wout_ref, bout_ref, w1_ref, b1_ref, w2_ref,
             b2_ref, g1_ref, be1_ref, g2_ref, be2_ref, out_ref):
    s0 = (jnp.dot(smp_ref[...], wout_ref[...], preferred_element_type=jnp.float32)
          + bout_ref[...] + src_ref[...])
    x1 = _ln(s0, g1_ref[...], be1_ref[...])
    h = jnp.maximum(
        jnp.dot(x1, w1_ref[...], preferred_element_type=jnp.float32) + b1_ref[...],
        0.0)
    ff = jnp.dot(h, w2_ref[...], preferred_element_type=jnp.float32) + b2_ref[...]
    out_ref[...] = _ln(x1 + ff, g2_ref[...], be2_ref[...])


NW = 32
QPW = NQ // NW


QC = 2
NCH = QPW // QC


def _sc_body(table_hbm, idx_hbm, w_hbm, out_hbm,
             idxA, idxB, wA, wB, rowsA, rowsB, outv,
             semGA, semGB, semIA, semIB, semWA, semWB):
    cid = lax.axis_index("c")
    sid = lax.axis_index("s")
    wid = sid * 2 + cid
    cbase = wid * NCH

    def fire(idxX, rowsX, semX):
        for qs in range(QC):
            for cc in range(4):
                pltpu.async_copy(
                    table_hbm.at[idxX.at[qs, cc]],
                    rowsX.at[pl.ds(qs * 512 + cc * 128, 128)], semX)

    def drain_rows(rowsX, semX):
        pltpu.make_async_copy(
            table_hbm.at[pl.ds(0, QC * 512)], rowsX, semX).wait()

    def drain_idx(idxX, semX):
        pltpu.make_async_copy(idx_hbm.at[pl.ds(0, QC)], idxX, semX).wait()

    def drain_w(wX, semX):
        pltpu.make_async_copy(w_hbm.at[pl.ds(0, QC)], wX, semX).wait()

    def compute(g, wX, rowsX):
        for qs in range(QC):
            for h in range(NH):
                acc0 = jnp.zeros((16,), jnp.float32)
                acc1 = jnp.zeros((16,), jnp.float32)
                for cc in range(4):
                    off = cc * 128 + h * 16
                    wvec = wX[qs, pl.ds(off, 16)]
                    rbase = qs * 512 + off
                    for t in range(16):
                        ws = wvec[t]
                        acc0 = acc0 + ws * rowsX[rbase + t, 0:16]
                        acc1 = acc1 + ws * rowsX[rbase + t, 16:32]
                outv[qs * NH + h, 0:16] = acc0
                outv[qs * NH + h, 16:32] = acc1
        pltpu.sync_copy(
            outv, out_hbm.at[pl.ds((cbase + g) * (QC * NH), QC * NH)])

    q0 = cbase * QC
    pltpu.sync_copy(idx_hbm.at[pl.ds(q0, QC)], idxA)
    fire(idxA, rowsA, semGA)
    pltpu.async_copy(idx_hbm.at[pl.ds(q0 + QC, QC)], idxB, semIB)
    pltpu.async_copy(w_hbm.at[pl.ds(q0, QC)], wA, semWA)
    pltpu.async_copy(w_hbm.at[pl.ds(q0 + QC, QC)], wB, semWB)

    bufsA = (idxA, wA, rowsA, semGA, semIA, semWA)
    bufsB = (idxB, wB, rowsB, semGB, semIB, semWB)

    def pair(i, carry):
        for par, (iX, wX, rX, sGX, sIX, sWX), (iY, wY, rY, sGY, sIY, sWY) in (
                (0, bufsA, bufsB), (1, bufsB, bufsA)):
            g = 2 * i + par
            drain_idx(iY, sIY)
            fire(iY, rY, sGY)
            qpre = (cbase + jnp.minimum(g + 2, NCH - 1)) * QC
            pltpu.async_copy(idx_hbm.at[pl.ds(qpre, QC)], iX, sIX)
            drain_rows(rX, sGX)
            drain_w(wX, sWX)
            compute(g, wX, rX)
            pltpu.async_copy(w_hbm.at[pl.ds(qpre, QC)], wX, sWX)
        return carry

    lax.fori_loop(0, NCH // 2, pair, 0)

    drain_rows(rowsA, semGA)
    drain_idx(idxB, semIB)
    drain_w(wA, semWA)
    drain_w(wB, semWB)


def _sample_sc(table, idx3, wts):
    return pl.kernel(
        _sc_body,
        out_type=jax.ShapeDtypeStruct((NROWS, DH), jnp.float32),
        mesh=plsc.VectorSubcoreMesh(core_axis_name="c", subcore_axis_name="s"),
        scratch_types=[
            pltpu.VMEM((QC, 4, 128), jnp.int32),
            pltpu.VMEM((QC, 4, 128), jnp.int32),
            pltpu.VMEM((QC, 512), jnp.float32),
            pltpu.VMEM((QC, 512), jnp.float32),
            pltpu.VMEM((QC * 512, DH), jnp.float32),
            pltpu.VMEM((QC * 512, DH), jnp.float32),
            pltpu.VMEM((QC * NH, DH), jnp.float32),
            pltpu.SemaphoreType.DMA,
            pltpu.SemaphoreType.DMA,
            pltpu.SemaphoreType.DMA,
            pltpu.SemaphoreType.DMA,
            pltpu.SemaphoreType.DMA,
            pltpu.SemaphoreType.DMA,
        ],
        compiler_params=pltpu.CompilerParams(use_tc_tiling_on_sc=False),
    )(table, idx3, wts)


def _stage1_call(src2, pos2, rpb, wval, bval, woffp, boffp, wattn, battn):
    consts_f = [jnp.asarray(_G), jnp.asarray(_WV[None, :]),
                jnp.asarray(_HV[None, :])]
    consts_i = [jnp.asarray(_W8V[None, :]), jnp.asarray(_BASE8[None, :])]
    row_spec = pl.BlockSpec((QB1, 256), lambda i: (i, 0))
    full = lambda shape: pl.BlockSpec(shape, lambda i: (0,) * len(shape))
    return pl.pallas_call(
        _k1_body,
        grid=(GRID1,),
        in_specs=[
            row_spec, row_spec, row_spec,
            full((C, C)), full((1, C)),
            full((C, C)), full((1, C)),
            full((C, 128)), full((1, 128)),
            full((128, 128)), full((1, 128)), full((1, 128)),
            full((1, 128)), full((1, 128)),
        ],
        out_specs=[
            pl.BlockSpec((QB1, C), lambda i: (i, 0)),
            pl.BlockSpec((QB1, 512), lambda i: (i, 0)),
            pl.BlockSpec((QB1, 512), lambda i: (i, 0)),
        ],
        out_shape=[
            jax.ShapeDtypeStruct((NQ, C), jnp.float32),
            jax.ShapeDtypeStruct((NQ, 512), jnp.int32),
            jax.ShapeDtypeStruct((NQ, 512), jnp.float32),
        ],
    )(src2, pos2, rpb, wval, bval, woffp, boffp, wattn, battn, *consts_f,
      *consts_i)


def _stage2_call(smp2, src2, wout, bout, w1, b1, w2, b2, g1, be1, g2, be2):
    row_spec = pl.BlockSpec((QB1, 256), lambda i: (i, 0))
    full = lambda shape: pl.BlockSpec(shape, lambda i: (0,) * len(shape))
    return pl.pallas_call(
        _k2_body,
        grid=(GRID1,),
        in_specs=[
            row_spec, row_spec,
            full((C, C)), full((1, C)),
            full((C, D_FFN)), full((1, D_FFN)),
            full((D_FFN, C)), full((1, C)),
            full((1, C)), full((1, C)), full((1, C)), full((1, C)),
        ],
        out_specs=pl.BlockSpec((QB1, C), lambda i: (i, 0)),
        out_shape=jax.ShapeDtypeStruct((NQ, C), jnp.float32),
    )(smp2, src2, wout, bout, w1, b1, w2, b2, g1, be1, g2, be2)


def kernel(src, pos, reference_points, spatial_shapes, level_start_index,
           padding_mask, W_off, b_off, W_attn, b_attn, W_val, b_val, W_out,
           b_out, W1, b1, W2, b2, g1, be1, g2, be2):
    del spatial_shapes, level_start_index, padding_mask

    rp = reference_points
    rpx = jnp.broadcast_to(rp[:, :, None, :, None, 0], (B, NQ, NH, NL, NP))
    rpy = jnp.broadcast_to(rp[:, :, None, :, None, 1], (B, NQ, NH, NL, NP))
    rpb = jnp.concatenate(
        [rpx.reshape(B, NQ, 128), rpy.reshape(B, NQ, 128)], axis=-1)

    woffp = W_off.reshape(C, NH, NL, NP, 2).transpose(0, 4, 1, 2, 3).reshape(C, 256)
    boffp = b_off.reshape(NH, NL, NP, 2).transpose(3, 0, 1, 2).reshape(1, 256)

    outs = []
    for b in range(B):
        srcb = src[b]
        val, idx, wts = _stage1_call(
            srcb, pos[b], rpb[b], W_val, b_val.reshape(1, C), woffp, boffp,
            W_attn, b_attn.reshape(1, 128))
        table = val.reshape(NROWS, DH)
        idx3 = idx.reshape(NQ, 4, 128)
        sampled = _sample_sc(table, idx3, wts).reshape(NQ, C)
        outs.append(_stage2_call(
            sampled, srcb, W_out, b_out.reshape(1, C), W1,
            b1.reshape(1, D_FFN), W2, b2.reshape(1, C), g1.reshape(1, C),
            be1.reshape(1, C), g2.reshape(1, C), be2.reshape(1, C)))
    return jnp.stack(outs)
